# Initial kernel scaffold; baseline (speedup 1.0000x reference)
import jax
import jax.numpy as jnp
from jax.experimental import pallas as pl


def kernel(x, gate_w, ew1, ew2, eow, eob, sw1, sw2):
    raise NotImplementedError("write your pallas kernel here")


# speedup vs baseline: 1.1843x; 1.1843x over previous
import functools

import jax
import jax.numpy as jnp
from jax.experimental import pallas as pl
from jax.experimental.pallas import tpu as pltpu

DIM = 1024
E = 8
NS = 2
B = 2
S = 2048
T = B * S
H_E = DIM * 4
H_S = DIM * 2

TB = 1024
NT = T // TB
CH = 512
NC_E = H_E // CH
NC_S = H_S // CH


def _router_body(x_ref, gwt_ref, eob_ref, wt_ref, bias_ref, lb_ref):
    x = x_ref[...]
    logits = jnp.dot(x, gwt_ref[...], preferred_element_type=jnp.float32)
    iota = jax.lax.broadcasted_iota(jnp.int32, (T, E), 1)
    l0 = jnp.max(logits, axis=1, keepdims=True)
    i0 = jnp.min(jnp.where(logits == l0, iota, E), axis=1, keepdims=True)
    masked = jnp.where(iota == i0, -jnp.inf, logits)
    l1 = jnp.max(masked, axis=1, keepdims=True)
    i1 = jnp.min(jnp.where(masked == l1, iota, E), axis=1, keepdims=True)
    g0 = jax.nn.sigmoid(l0 - l1)
    g1 = 1.0 - g0
    onehot0 = (iota == i0).astype(jnp.float32)
    onehot1 = (iota == i1).astype(jnp.float32)
    w = g0 * onehot0 + g1 * onehot1
    wt_ref[...] = w.T[:, None, :]
    bias_ref[...] = jnp.dot(w, eob_ref[...], preferred_element_type=jnp.float32)
    counts = jnp.sum(onehot0, axis=0)
    total = jnp.sum(counts)
    lb = jnp.mean((counts / total - 1.0 / E) ** 2)
    lb_ref[...] = lb.reshape(1, 1)


def _router(xf, gate_w, eob):
    wt, bias, lb = pl.pallas_call(
        _router_body,
        out_shape=(
            jax.ShapeDtypeStruct((E, 1, T), jnp.float32),
            jax.ShapeDtypeStruct((T, DIM), jnp.float32),
            jax.ShapeDtypeStruct((1, 1), jnp.float32),
        ),
    )(xf, gate_w.T, eob)
    return wt, bias, lb


def _w2e_body(ew2_ref, eow_ref, out_ref):
    out_ref[...] = jnp.dot(ew2_ref[0], eow_ref[0],
                           preferred_element_type=jnp.float32)[None]


def _w2e(ew2, eow):
    rb = 1024
    return pl.pallas_call(
        _w2e_body,
        grid=(E, H_E // rb),
        in_specs=[
            pl.BlockSpec((1, rb, DIM), lambda e, r: (e, r, 0)),
            pl.BlockSpec((1, DIM, DIM), lambda e, r: (e, 0, 0)),
        ],
        out_specs=pl.BlockSpec((1, rb, DIM), lambda e, r: (e, r, 0)),
        out_shape=jax.ShapeDtypeStruct((E, H_E, DIM), jnp.float32),
    )(ew2, eow)


def _routed_body(x_ref, w1a_ref, w1g_ref, w2e_ref, wt_ref, out_ref):
    e = pl.program_id(0)
    c = pl.program_id(1)
    t = pl.program_id(2)

    @pl.when((e == 0) & (c == 0) & (t == 0))
    def _():
        out_ref[...] = jnp.zeros_like(out_ref)

    x = x_ref[...]
    a = jnp.dot(x, w1a_ref[0], preferred_element_type=jnp.float32)
    g = jnp.dot(x, w1g_ref[0], preferred_element_type=jnp.float32)
    s = a * jax.nn.sigmoid(a) * g
    z = jnp.dot(s, w2e_ref[0], preferred_element_type=jnp.float32)
    wcol = wt_ref[0, 0][:, None]
    out_ref[pl.ds(t * TB, TB), :] = out_ref[pl.ds(t * TB, TB), :] + wcol * z


def _routed(xf, ew1, w2e, wt):
    return pl.pallas_call(
        _routed_body,
        grid=(E, NC_E, NT),
        in_specs=[
            pl.BlockSpec((TB, DIM), lambda e, c, t: (t, 0)),
            pl.BlockSpec((1, DIM, CH), lambda e, c, t: (e, 0, c)),
            pl.BlockSpec((1, DIM, CH), lambda e, c, t: (e, 0, c + H_E // CH)),
            pl.BlockSpec((1, CH, DIM), lambda e, c, t: (e, c, 0)),
            pl.BlockSpec((1, 1, TB), lambda e, c, t: (e, 0, t)),
        ],
        out_specs=pl.BlockSpec((T, DIM), lambda e, c, t: (0, 0)),
        out_shape=jax.ShapeDtypeStruct((T, DIM), jnp.float32),
    )(xf, ew1, ew1, w2e, wt)


def _shared_body(x_ref, w1a_ref, w1g_ref, w2_ref, out_ref):
    s_i = pl.program_id(0)
    c = pl.program_id(1)
    t = pl.program_id(2)

    @pl.when((s_i == 0) & (c == 0) & (t == 0))
    def _():
        out_ref[...] = jnp.zeros_like(out_ref)

    x = x_ref[...]
    a = jnp.dot(x, w1a_ref[0], preferred_element_type=jnp.float32)
    g = jnp.dot(x, w1g_ref[0], preferred_element_type=jnp.float32)
    act = a * jax.nn.sigmoid(a) * g
    z = jnp.dot(act, w2_ref[0], preferred_element_type=jnp.float32)
    out_ref[pl.ds(t * TB, TB), :] = out_ref[pl.ds(t * TB, TB), :] + z * (1.0 / NS)


def _shared(xf, sw1, sw2):
    return pl.pallas_call(
        _shared_body,
        grid=(NS, NC_S, NT),
        in_specs=[
            pl.BlockSpec((TB, DIM), lambda s, c, t: (t, 0)),
            pl.BlockSpec((1, DIM, CH), lambda s, c, t: (s, 0, c)),
            pl.BlockSpec((1, DIM, CH), lambda s, c, t: (s, 0, c + H_S // CH)),
            pl.BlockSpec((1, CH, DIM), lambda s, c, t: (s, c, 0)),
        ],
        out_specs=pl.BlockSpec((T, DIM), lambda s, c, t: (0, 0)),
        out_shape=jax.ShapeDtypeStruct((T, DIM), jnp.float32),
    )(xf, sw1, sw1, sw2)


def kernel(x, gate_w, ew1, ew2, eow, eob, sw1, sw2):
    xf = x.reshape(T, DIM)
    wt, bias, lb = _router(xf, gate_w, eob)
    w2e = _w2e(ew2, eow)
    routed = _routed(xf, ew1, w2e, wt)
    shared = _shared(xf, sw1, sw2)
    out = (routed + shared + bias).reshape(B, S, DIM)
    return (out, lb[0, 0])


# speedup vs baseline: 1.2094x; 1.0211x over previous
import functools

import jax
import jax.numpy as jnp
from jax import lax
from jax.experimental import pallas as pl
from jax.experimental.pallas import tpu as pltpu
from jax.experimental.pallas import tpu_sc as plsc

DIM = 1024
E = 8
NS = 2
B = 2
S = 2048
T = B * S
A = 2 * T
H_E = DIM * 4
H_S = DIM * 2

BLK = 256
NB = 40
NP = NB * BLK

R = 128
G = A // R

JB = 512
NJB = NP // JB

TB = 1024
NT = T // TB
CH = 512
NC_S = H_S // CH

ECH = 1024
NC_E = H_E // ECH

NWORK = 32
WROWS = NP // NWORK
WCH = 64
NWCH = WROWS // WCH

TRASH = A


def _router_body(logits_ref, lb_ref, aux_ref, be_ref):
    f32 = jnp.float32
    logits = logits_ref[...]
    iota_e = lax.broadcasted_iota(jnp.int32, (T, E), 1)
    l0 = jnp.max(logits, axis=1, keepdims=True)
    i0 = jnp.min(jnp.where(logits == l0, iota_e, E), axis=1, keepdims=True)
    masked = jnp.where(iota_e == i0, -jnp.inf, logits)
    l1 = jnp.max(masked, axis=1, keepdims=True)
    i1 = jnp.min(jnp.where(masked == l1, iota_e, E), axis=1, keepdims=True)
    g0 = jax.nn.sigmoid(l0 - l1)
    g1 = 1.0 - g0
    onehot0 = (iota_e == i0).astype(f32)
    onehot1 = (iota_e == i1).astype(f32)

    counts0 = jnp.sum(onehot0, axis=0)
    total = jnp.sum(counts0)
    lb_ref[...] = jnp.mean((counts0 / total - 1.0 / E) ** 2).reshape(1, 1)

    m = jnp.concatenate([onehot0, onehot1], axis=0)
    m3 = m.reshape(G, R, E)
    ir = lax.broadcasted_iota(jnp.int32, (R, R), 0)
    ic = lax.broadcasted_iota(jnp.int32, (R, R), 1)
    lr_excl = (ic < ir).astype(f32)
    lrb = jnp.broadcast_to(lr_excl[None], (G, R, R))
    within = lax.dot_general(lrb, m3, (((2,), (1,)), ((0,), (0,))),
                             preferred_element_type=f32)
    tot = jnp.sum(m3, axis=1)
    ig_r = lax.broadcasted_iota(jnp.int32, (G, G), 0)
    ig_c = lax.broadcasted_iota(jnp.int32, (G, G), 1)
    lg_excl = (ig_c < ig_r).astype(f32)
    base = jnp.dot(lg_excl, tot, preferred_element_type=f32)
    rank3 = within + base[:, None, :]
    rank = jnp.sum(rank3 * m3, axis=2).reshape(A, 1)

    ones_g = jnp.ones((G, 1), f32)
    counts_col = lax.dot_general(tot, ones_g, (((0,), (0,)), ((), ())),
                                 preferred_element_type=f32)
    pc = jnp.floor((counts_col + (BLK - 1)) * (1.0 / BLK)) * BLK
    ie_r = lax.broadcasted_iota(jnp.int32, (E, E), 0)
    ie_c = lax.broadcasted_iota(jnp.int32, (E, E), 1)
    le_excl = (ie_c < ie_r).astype(f32)
    le_incl = (ie_c <= ie_r).astype(f32)
    po = jnp.dot(le_excl, pc, preferred_element_type=f32)
    cb = jnp.dot(le_incl, pc * (1.0 / BLK), preferred_element_type=f32)

    nbv = lax.broadcasted_iota(jnp.int32, (1, NB), 1).astype(f32)
    be = jnp.sum((cb <= nbv).astype(f32), axis=0, keepdims=True)
    be_ref[...] = jnp.minimum(be, E - 1).astype(jnp.int32)

    dpp = jnp.dot(m, po, preferred_element_type=f32) + rank
    gates = jnp.concatenate([g0, g1], axis=0)
    aux_ref[...] = jnp.concatenate([dpp, gates], axis=1)


def _invperm_body(aux_ref, src_ref, wg_ref, dest_ref):
    f32 = jnp.float32
    jb = pl.program_id(0)
    dpp = aux_ref[:, 0:1]
    gates = aux_ref[:, 1:2]
    ia = lax.broadcasted_iota(jnp.int32, (A, 1), 0)
    slot_i = (ia >= T).astype(jnp.int32)
    tok = (ia - slot_i * T).astype(f32)
    t_hi = jnp.floor(tok * (1.0 / 64.0))
    t_lo = tok - t_hi * 64.0
    payload = jnp.concatenate([
        t_hi, t_lo, gates, slot_i.astype(f32), jnp.ones((A, 1), f32),
    ], axis=1)
    jrow = jb * JB + lax.broadcasted_iota(jnp.int32, (1, JB), 1).astype(f32)
    p = (dpp == jrow).astype(f32)
    res = lax.dot_general(p, payload, (((0,), (0,)), ((), ())),
                          preferred_element_type=f32,
                          precision=lax.Precision.HIGHEST)
    srcv = res[:, 0] * 64.0 + res[:, 1]
    gate = res[:, 2]
    slot = res[:, 3]
    valid = res[:, 4]
    dest = jnp.where(valid > 0.5, slot * T + srcv, float(TRASH))
    src_ref[0, 0, :] = srcv.astype(jnp.int32)
    wg_ref[0, 0, :] = gate
    dest_ref[0, 0, :] = dest.astype(jnp.int32)


def _router(logits):
    lb, aux, be = pl.pallas_call(
        _router_body,
        out_shape=(
            jax.ShapeDtypeStruct((1, 1), jnp.float32),
            jax.ShapeDtypeStruct((A, 2), jnp.float32),
            jax.ShapeDtypeStruct((1, NB), jnp.int32),
        ),
    )(logits)
    src, wg, dest = pl.pallas_call(
        _invperm_body,
        grid=(NJB,),
        in_specs=[pl.BlockSpec((A, 2), lambda jb: (0, 0))],
        out_specs=(
            pl.BlockSpec((1, 1, JB), lambda jb: (jb, 0, 0)),
            pl.BlockSpec((1, 1, JB), lambda jb: (jb, 0, 0)),
            pl.BlockSpec((1, 1, JB), lambda jb: (jb, 0, 0)),
        ),
        out_shape=(
            jax.ShapeDtypeStruct((NJB, 1, JB), jnp.int32),
            jax.ShapeDtypeStruct((NJB, 1, JB), jnp.float32),
            jax.ShapeDtypeStruct((NJB, 1, JB), jnp.int32),
        ),
    )(aux)
    return lb, src.reshape(NP), wg.reshape(NP, 1), dest.reshape(NP), \
        be.reshape(NB)


def _sc_mesh():
    return plsc.VectorSubcoreMesh(core_axis_name="c", subcore_axis_name="s",
                                  num_cores=2, num_subcores=16)


def _worker_id():
    return lax.axis_index("s") * 2 + lax.axis_index("c")


def _sc_gather(xf, src3):
    @functools.partial(
        pl.kernel,
        out_type=jax.ShapeDtypeStruct((NP, DIM), jnp.float32),
        mesh=_sc_mesh(),
        scratch_types=[
            pltpu.VMEM((NWCH, WCH), jnp.int32),
            pltpu.VMEM((WCH, DIM), jnp.float32),
            pltpu.SemaphoreType.DMA,
        ],
    )
    def k(x_hbm, src_hbm, xg_hbm, idx_v, rows_v, sem):
        wid = _worker_id()
        base = wid * WROWS
        pltpu.sync_copy(src_hbm.at[wid], idx_v)
        for ci in range(NWCH):
            pltpu.async_copy(x_hbm.at[idx_v.at[ci]], rows_v, sem).wait()
            pltpu.sync_copy(rows_v, xg_hbm.at[pl.ds(base + ci * WCH, WCH)])

    return k(xf, src3)


def _sc_scatter(zw, dest3):
    @functools.partial(
        pl.kernel,
        out_type=jax.ShapeDtypeStruct((A + 8, DIM), jnp.float32),
        mesh=_sc_mesh(),
        scratch_types=[
            pltpu.VMEM((NWCH, WCH), jnp.int32),
            pltpu.VMEM((WCH, DIM), jnp.float32),
            pltpu.SemaphoreType.DMA,
        ],
    )
    def k(zw_hbm, dest_hbm, out_hbm, idx_v, rows_v, sem):
        wid = _worker_id()
        base = wid * WROWS
        pltpu.sync_copy(dest_hbm.at[wid], idx_v)
        for ci in range(NWCH):
            pltpu.sync_copy(zw_hbm.at[pl.ds(base + ci * WCH, WCH)], rows_v)
            pltpu.async_copy(rows_v, out_hbm.at[idx_v.at[ci]], sem).wait()

    return k(zw, dest3)


def _grouped_body_first(be_ref, xg_ref, w1a_ref, w1g_ref, w2_ref, y_ref):
    x = xg_ref[...]
    a = jnp.dot(x, w1a_ref[0], preferred_element_type=jnp.float32)
    g = jnp.dot(x, w1g_ref[0], preferred_element_type=jnp.float32)
    s = a * jax.nn.sigmoid(a) * g
    y_ref[...] = jnp.dot(s, w2_ref[0], preferred_element_type=jnp.float32)


def _grouped_body_acc(be_ref, xg_ref, w1a_ref, w1g_ref, w2_ref, yin_ref, y_ref):
    x = xg_ref[...]
    a = jnp.dot(x, w1a_ref[0], preferred_element_type=jnp.float32)
    g = jnp.dot(x, w1g_ref[0], preferred_element_type=jnp.float32)
    s = a * jax.nn.sigmoid(a) * g
    y_ref[...] = yin_ref[...] + jnp.dot(s, w2_ref[0],
                                        preferred_element_type=jnp.float32)


def _grouped(xg, ew1, ew2, be):
    def chunk_call(c, y_prev):
        first = y_prev is None
        in_specs = [
            pl.BlockSpec((BLK, DIM), lambda nb, be_ref: (nb, 0)),
            pl.BlockSpec((1, DIM, ECH), lambda nb, be_ref: (be_ref[nb], 0, c)),
            pl.BlockSpec((1, DIM, ECH),
                         lambda nb, be_ref: (be_ref[nb], 0, c + NC_E)),
            pl.BlockSpec((1, ECH, DIM), lambda nb, be_ref: (be_ref[nb], c, 0)),
        ]
        args = [be, xg, ew1, ew1, ew2]
        if not first:
            in_specs.append(pl.BlockSpec((BLK, DIM), lambda nb, be_ref: (nb, 0)))
            args.append(y_prev)
        grid_spec = pltpu.PrefetchScalarGridSpec(
            num_scalar_prefetch=1,
            grid=(NB,),
            in_specs=in_specs,
            out_specs=pl.BlockSpec((BLK, DIM), lambda nb, be_ref: (nb, 0)),
        )
        return pl.pallas_call(
            _grouped_body_first if first else _grouped_body_acc,
            grid_spec=grid_spec,
            out_shape=jax.ShapeDtypeStruct((NP, DIM), jnp.float32),
        )(*args)

    y = chunk_call(0, None)
    for c in range(1, NC_E):
        y = chunk_call(c, y)
    return y


def _outproj_body(be_ref, y_ref, eow_ref, eob_ref, wg_ref, zw_ref):
    z = jnp.dot(y_ref[...], eow_ref[0], preferred_element_type=jnp.float32)
    zw_ref[...] = (z + eob_ref[0]) * wg_ref[...]


def _outproj(y, eow, eob, wg, be):
    grid_spec = pltpu.PrefetchScalarGridSpec(
        num_scalar_prefetch=1,
        grid=(NB,),
        in_specs=[
            pl.BlockSpec((BLK, DIM), lambda nb, be_ref: (nb, 0)),
            pl.BlockSpec((1, DIM, DIM), lambda nb, be_ref: (be_ref[nb], 0, 0)),
            pl.BlockSpec((1, 1, DIM), lambda nb, be_ref: (be_ref[nb], 0, 0)),
            pl.BlockSpec((BLK, 1), lambda nb, be_ref: (nb, 0)),
        ],
        out_specs=pl.BlockSpec((BLK, DIM), lambda nb, be_ref: (nb, 0)),
    )
    return pl.pallas_call(
        _outproj_body,
        grid_spec=grid_spec,
        out_shape=jax.ShapeDtypeStruct((NP, DIM), jnp.float32),
    )(be, y, eow, eob.reshape(E, 1, DIM), wg)


def _shared_body(x_ref, w1a_ref, w1g_ref, w2_ref, out_ref):
    s_i = pl.program_id(0)
    c = pl.program_id(1)
    t = pl.program_id(2)

    @pl.when((s_i == 0) & (c == 0) & (t == 0))
    def _():
        out_ref[...] = jnp.zeros_like(out_ref)

    x = x_ref[...]
    a = jnp.dot(x, w1a_ref[0], preferred_element_type=jnp.float32)
    g = jnp.dot(x, w1g_ref[0], preferred_element_type=jnp.float32)
    act = a * jax.nn.sigmoid(a) * g
    z = jnp.dot(act, w2_ref[0], preferred_element_type=jnp.float32)
    out_ref[pl.ds(t * TB, TB), :] = out_ref[pl.ds(t * TB, TB), :] + z * (1.0 / NS)


def _shared(xf, sw1, sw2):
    return pl.pallas_call(
        _shared_body,
        grid=(NS, NC_S, NT),
        in_specs=[
            pl.BlockSpec((TB, DIM), lambda s, c, t: (t, 0)),
            pl.BlockSpec((1, DIM, CH), lambda s, c, t: (s, 0, c)),
            pl.BlockSpec((1, DIM, CH), lambda s, c, t: (s, 0, c + H_S // CH)),
            pl.BlockSpec((1, CH, DIM), lambda s, c, t: (s, c, 0)),
        ],
        out_specs=pl.BlockSpec((T, DIM), lambda s, c, t: (0, 0)),
        out_shape=jax.ShapeDtypeStruct((T, DIM), jnp.float32),
    )(xf, sw1, sw1, sw2)


def _combine_body(sh_ref, r0_ref, r1_ref, out_ref):
    out_ref[...] = sh_ref[...] + r0_ref[...] + r1_ref[...]


def _combine(shared, routed2):
    return pl.pallas_call(
        _combine_body,
        grid=(NT,),
        in_specs=[
            pl.BlockSpec((TB, DIM), lambda t: (t, 0)),
            pl.BlockSpec((TB, DIM), lambda t: (t, 0)),
            pl.BlockSpec((TB, DIM), lambda t: (t + T // TB, 0)),
        ],
        out_specs=pl.BlockSpec((TB, DIM), lambda t: (t, 0)),
        out_shape=jax.ShapeDtypeStruct((T, DIM), jnp.float32),
    )(shared, routed2, routed2)


def kernel(x, gate_w, ew1, ew2, eow, eob, sw1, sw2):
    xf = x.reshape(T, DIM)
    gate_logits = jnp.einsum('bsd,ed->bse', x, gate_w)
    lb, src, wg, dest, be = _router(gate_logits.reshape(T, E))
    xg = _sc_gather(xf, src.reshape(NWORK, NWCH, WCH))
    y = _grouped(xg, ew1, ew2, be)
    zw = _outproj(y, eow, eob, wg, be)
    routed2 = _sc_scatter(zw, dest.reshape(NWORK, NWCH, WCH))
    shared = _shared(xf, sw1, sw2)
    out = _combine(shared, routed2).reshape(B, S, DIM)
    return (out, lb[0, 0])


# speedup vs baseline: 1.2112x; 1.0015x over previous
import functools

import jax
import jax.numpy as jnp
from jax import lax
from jax.experimental import pallas as pl
from jax.experimental.pallas import tpu as pltpu
from jax.experimental.pallas import tpu_sc as plsc

DIM = 1024
E = 8
NS = 2
B = 2
S = 2048
T = B * S
A = 2 * T
H_E = DIM * 4
H_S = DIM * 2

BLK = 256
NB = 40
NP = NB * BLK

R = 128
G = A // R

JB = 512
NJB = NP // JB

TB = 1024
NT = T // TB
CH = 512
NC_S = H_S // CH

ECH = 1024
NC_E = H_E // ECH

NWORK = 32
WROWS = NP // NWORK
WCH = 40
NWCH = WROWS // WCH

TRASH = A


def _router_body(logits_ref, lb_ref, aux_ref, be_ref):
    f32 = jnp.float32
    logits = logits_ref[...]
    iota_e = lax.broadcasted_iota(jnp.int32, (T, E), 1)
    l0 = jnp.max(logits, axis=1, keepdims=True)
    i0 = jnp.min(jnp.where(logits == l0, iota_e, E), axis=1, keepdims=True)
    masked = jnp.where(iota_e == i0, -jnp.inf, logits)
    l1 = jnp.max(masked, axis=1, keepdims=True)
    i1 = jnp.min(jnp.where(masked == l1, iota_e, E), axis=1, keepdims=True)
    g0 = jax.nn.sigmoid(l0 - l1)
    g1 = 1.0 - g0
    onehot0 = (iota_e == i0).astype(f32)
    onehot1 = (iota_e == i1).astype(f32)

    counts0 = jnp.sum(onehot0, axis=0)
    total = jnp.sum(counts0)
    lb_ref[...] = jnp.mean((counts0 / total - 1.0 / E) ** 2).reshape(1, 1)

    m = jnp.concatenate([onehot0, onehot1], axis=0)
    m3 = m.reshape(G, R, E)
    ir = lax.broadcasted_iota(jnp.int32, (R, R), 0)
    ic = lax.broadcasted_iota(jnp.int32, (R, R), 1)
    lr_excl = (ic < ir).astype(f32)
    lrb = jnp.broadcast_to(lr_excl[None], (G, R, R))
    within = lax.dot_general(lrb, m3, (((2,), (1,)), ((0,), (0,))),
                             preferred_element_type=f32)
    tot = jnp.sum(m3, axis=1)
    ig_r = lax.broadcasted_iota(jnp.int32, (G, G), 0)
    ig_c = lax.broadcasted_iota(jnp.int32, (G, G), 1)
    lg_excl = (ig_c < ig_r).astype(f32)
    base = jnp.dot(lg_excl, tot, preferred_element_type=f32)
    rank3 = within + base[:, None, :]
    rank = jnp.sum(rank3 * m3, axis=2).reshape(A, 1)

    ones_g = jnp.ones((G, 1), f32)
    counts_col = lax.dot_general(tot, ones_g, (((0,), (0,)), ((), ())),
                                 preferred_element_type=f32)
    pc = jnp.floor((counts_col + (BLK - 1)) * (1.0 / BLK)) * BLK
    ie_r = lax.broadcasted_iota(jnp.int32, (E, E), 0)
    ie_c = lax.broadcasted_iota(jnp.int32, (E, E), 1)
    le_excl = (ie_c < ie_r).astype(f32)
    le_incl = (ie_c <= ie_r).astype(f32)
    po = jnp.dot(le_excl, pc, preferred_element_type=f32)
    cb = jnp.dot(le_incl, pc * (1.0 / BLK), preferred_element_type=f32)

    nbv = lax.broadcasted_iota(jnp.int32, (1, NB), 1).astype(f32)
    be = jnp.sum((cb <= nbv).astype(f32), axis=0, keepdims=True)
    be_ref[...] = jnp.minimum(be, E - 1).astype(jnp.int32)

    dpp = jnp.dot(m, po, preferred_element_type=f32) + rank
    gates = jnp.concatenate([g0, g1], axis=0)
    aux_ref[...] = jnp.concatenate([dpp, gates], axis=1)


def _invperm_body(aux_ref, src_ref, wg_ref, dest_ref):
    f32 = jnp.float32
    jb = pl.program_id(0)
    dpp = aux_ref[:, 0:1]
    gates = aux_ref[:, 1:2]
    ia = lax.broadcasted_iota(jnp.int32, (A, 1), 0)
    slot_i = (ia >= T).astype(jnp.int32)
    tok = (ia - slot_i * T).astype(f32)
    t_hi = jnp.floor(tok * (1.0 / 64.0))
    t_lo = tok - t_hi * 64.0
    payload = jnp.concatenate([
        t_hi, t_lo, gates, slot_i.astype(f32), jnp.ones((A, 1), f32),
    ], axis=1)
    jrow = jb * JB + lax.broadcasted_iota(jnp.int32, (1, JB), 1).astype(f32)
    p = (dpp == jrow).astype(f32)
    res = lax.dot_general(p, payload, (((0,), (0,)), ((), ())),
                          preferred_element_type=f32,
                          precision=lax.Precision.HIGHEST)
    srcv = res[:, 0] * 64.0 + res[:, 1]
    gate = res[:, 2]
    slot = res[:, 3]
    valid = res[:, 4]
    dest = jnp.where(valid > 0.5, slot * T + srcv, float(TRASH))
    src_ref[0, 0, :] = srcv.astype(jnp.int32)
    wg_ref[0, 0, :] = gate
    dest_ref[0, 0, :] = dest.astype(jnp.int32)


def _router(logits):
    lb, aux, be = pl.pallas_call(
        _router_body,
        out_shape=(
            jax.ShapeDtypeStruct((1, 1), jnp.float32),
            jax.ShapeDtypeStruct((A, 2), jnp.float32),
            jax.ShapeDtypeStruct((1, NB), jnp.int32),
        ),
    )(logits)
    src, wg, dest = pl.pallas_call(
        _invperm_body,
        grid=(NJB,),
        in_specs=[pl.BlockSpec((A, 2), lambda jb: (0, 0))],
        out_specs=(
            pl.BlockSpec((1, 1, JB), lambda jb: (jb, 0, 0)),
            pl.BlockSpec((1, 1, JB), lambda jb: (jb, 0, 0)),
            pl.BlockSpec((1, 1, JB), lambda jb: (jb, 0, 0)),
        ),
        out_shape=(
            jax.ShapeDtypeStruct((NJB, 1, JB), jnp.int32),
            jax.ShapeDtypeStruct((NJB, 1, JB), jnp.float32),
            jax.ShapeDtypeStruct((NJB, 1, JB), jnp.int32),
        ),
    )(aux)
    return lb, src.reshape(NP), wg.reshape(NP, 1), dest.reshape(NP), \
        be.reshape(NB)


def _sc_mesh():
    return plsc.VectorSubcoreMesh(core_axis_name="c", subcore_axis_name="s",
                                  num_cores=2, num_subcores=16)


def _worker_id():
    return lax.axis_index("s") * 2 + lax.axis_index("c")


def _sc_gather(xf, src3):
    @functools.partial(
        pl.kernel,
        out_type=jax.ShapeDtypeStruct((NP, DIM), jnp.float32),
        mesh=_sc_mesh(),
        scratch_types=[
            pltpu.VMEM((NWCH, WCH), jnp.int32),
            pltpu.VMEM((2, WCH, DIM), jnp.float32),
            pltpu.SemaphoreType.DMA,
            pltpu.SemaphoreType.DMA,
        ],
    )
    def k(x_hbm, src_hbm, xg_hbm, idx_v, rows_v, sem0, sem1):
        wid = _worker_id()
        base = wid * WROWS
        sems = (sem0, sem1)
        pltpu.sync_copy(src_hbm.at[wid], idx_v)
        hs = {0: pltpu.async_copy(x_hbm.at[idx_v.at[0]], rows_v.at[0], sems[0])}
        for ci in range(NWCH):
            b = ci % 2
            hs[ci].wait()
            if ci + 1 < NWCH:
                hs[ci + 1] = pltpu.async_copy(
                    x_hbm.at[idx_v.at[ci + 1]], rows_v.at[(ci + 1) % 2],
                    sems[(ci + 1) % 2])
            pltpu.sync_copy(rows_v.at[b], xg_hbm.at[pl.ds(base + ci * WCH, WCH)])

    return k(xf, src3)


def _sc_scatter(zw, dest3):
    @functools.partial(
        pl.kernel,
        out_type=jax.ShapeDtypeStruct((A + 8, DIM), jnp.float32),
        mesh=_sc_mesh(),
        scratch_types=[
            pltpu.VMEM((NWCH, WCH), jnp.int32),
            pltpu.VMEM((2, WCH, DIM), jnp.float32),
            pltpu.SemaphoreType.DMA,
            pltpu.SemaphoreType.DMA,
        ],
    )
    def k(zw_hbm, dest_hbm, out_hbm, idx_v, rows_v, sem0, sem1):
        wid = _worker_id()
        base = wid * WROWS
        sems = (sem0, sem1)
        pltpu.sync_copy(dest_hbm.at[wid], idx_v)
        hs = {}
        for ci in range(NWCH):
            b = ci % 2
            if ci >= 2:
                hs[ci - 2].wait()
            pltpu.sync_copy(zw_hbm.at[pl.ds(base + ci * WCH, WCH)], rows_v.at[b])
            hs[ci] = pltpu.async_copy(rows_v.at[b], out_hbm.at[idx_v.at[ci]],
                                      sems[b])
        hs[NWCH - 2].wait()
        hs[NWCH - 1].wait()

    return k(zw, dest3)


def _grouped_body_first(be_ref, xg_ref, w1a_ref, w1g_ref, w2_ref, y_ref):
    x = xg_ref[...]
    a = jnp.dot(x, w1a_ref[0], preferred_element_type=jnp.float32)
    g = jnp.dot(x, w1g_ref[0], preferred_element_type=jnp.float32)
    s = a * jax.nn.sigmoid(a) * g
    y_ref[...] = jnp.dot(s, w2_ref[0], preferred_element_type=jnp.float32)


def _grouped_body_acc(be_ref, xg_ref, w1a_ref, w1g_ref, w2_ref, yin_ref, y_ref):
    x = xg_ref[...]
    a = jnp.dot(x, w1a_ref[0], preferred_element_type=jnp.float32)
    g = jnp.dot(x, w1g_ref[0], preferred_element_type=jnp.float32)
    s = a * jax.nn.sigmoid(a) * g
    y_ref[...] = yin_ref[...] + jnp.dot(s, w2_ref[0],
                                        preferred_element_type=jnp.float32)


def _grouped(xg, ew1, ew2, be):
    def chunk_call(c, y_prev):
        first = y_prev is None
        in_specs = [
            pl.BlockSpec((BLK, DIM), lambda nb, be_ref: (nb, 0)),
            pl.BlockSpec((1, DIM, ECH), lambda nb, be_ref: (be_ref[nb], 0, c)),
            pl.BlockSpec((1, DIM, ECH),
                         lambda nb, be_ref: (be_ref[nb], 0, c + NC_E)),
            pl.BlockSpec((1, ECH, DIM), lambda nb, be_ref: (be_ref[nb], c, 0)),
        ]
        args = [be, xg, ew1, ew1, ew2]
        if not first:
            in_specs.append(pl.BlockSpec((BLK, DIM), lambda nb, be_ref: (nb, 0)))
            args.append(y_prev)
        grid_spec = pltpu.PrefetchScalarGridSpec(
            num_scalar_prefetch=1,
            grid=(NB,),
            in_specs=in_specs,
            out_specs=pl.BlockSpec((BLK, DIM), lambda nb, be_ref: (nb, 0)),
        )
        return pl.pallas_call(
            _grouped_body_first if first else _grouped_body_acc,
            grid_spec=grid_spec,
            out_shape=jax.ShapeDtypeStruct((NP, DIM), jnp.float32),
        )(*args)

    y = chunk_call(0, None)
    for c in range(1, NC_E):
        y = chunk_call(c, y)
    return y


def _outproj_body(be_ref, y_ref, eow_ref, eob_ref, wg_ref, zw_ref):
    z = jnp.dot(y_ref[...], eow_ref[0], preferred_element_type=jnp.float32)
    zw_ref[...] = (z + eob_ref[0]) * wg_ref[...]


def _outproj(y, eow, eob, wg, be):
    grid_spec = pltpu.PrefetchScalarGridSpec(
        num_scalar_prefetch=1,
        grid=(NB,),
        in_specs=[
            pl.BlockSpec((BLK, DIM), lambda nb, be_ref: (nb, 0)),
            pl.BlockSpec((1, DIM, DIM), lambda nb, be_ref: (be_ref[nb], 0, 0)),
            pl.BlockSpec((1, 1, DIM), lambda nb, be_ref: (be_ref[nb], 0, 0)),
            pl.BlockSpec((BLK, 1), lambda nb, be_ref: (nb, 0)),
        ],
        out_specs=pl.BlockSpec((BLK, DIM), lambda nb, be_ref: (nb, 0)),
    )
    return pl.pallas_call(
        _outproj_body,
        grid_spec=grid_spec,
        out_shape=jax.ShapeDtypeStruct((NP, DIM), jnp.float32),
    )(be, y, eow, eob.reshape(E, 1, DIM), wg)


def _shared_body(x_ref, w1a_ref, w1g_ref, w2_ref, out_ref):
    s_i = pl.program_id(0)
    c = pl.program_id(1)
    t = pl.program_id(2)

    @pl.when((s_i == 0) & (c == 0) & (t == 0))
    def _():
        out_ref[...] = jnp.zeros_like(out_ref)

    x = x_ref[...]
    a = jnp.dot(x, w1a_ref[0], preferred_element_type=jnp.float32)
    g = jnp.dot(x, w1g_ref[0], preferred_element_type=jnp.float32)
    act = a * jax.nn.sigmoid(a) * g
    z = jnp.dot(act, w2_ref[0], preferred_element_type=jnp.float32)
    out_ref[pl.ds(t * TB, TB), :] = out_ref[pl.ds(t * TB, TB), :] + z * (1.0 / NS)


def _shared(xf, sw1, sw2):
    return pl.pallas_call(
        _shared_body,
        grid=(NS, NC_S, NT),
        in_specs=[
            pl.BlockSpec((TB, DIM), lambda s, c, t: (t, 0)),
            pl.BlockSpec((1, DIM, CH), lambda s, c, t: (s, 0, c)),
            pl.BlockSpec((1, DIM, CH), lambda s, c, t: (s, 0, c + H_S // CH)),
            pl.BlockSpec((1, CH, DIM), lambda s, c, t: (s, c, 0)),
        ],
        out_specs=pl.BlockSpec((T, DIM), lambda s, c, t: (0, 0)),
        out_shape=jax.ShapeDtypeStruct((T, DIM), jnp.float32),
    )(xf, sw1, sw1, sw2)


def _combine_body(sh_ref, r0_ref, r1_ref, out_ref):
    out_ref[...] = sh_ref[...] + r0_ref[...] + r1_ref[...]


def _combine(shared, routed2):
    return pl.pallas_call(
        _combine_body,
        grid=(NT,),
        in_specs=[
            pl.BlockSpec((TB, DIM), lambda t: (t, 0)),
            pl.BlockSpec((TB, DIM), lambda t: (t, 0)),
            pl.BlockSpec((TB, DIM), lambda t: (t + T // TB, 0)),
        ],
        out_specs=pl.BlockSpec((TB, DIM), lambda t: (t, 0)),
        out_shape=jax.ShapeDtypeStruct((T, DIM), jnp.float32),
    )(shared, routed2, routed2)


def kernel(x, gate_w, ew1, ew2, eow, eob, sw1, sw2):
    xf = x.reshape(T, DIM)
    gate_logits = jnp.einsum('bsd,ed->bse', x, gate_w)
    lb, src, wg, dest, be = _router(gate_logits.reshape(T, E))
    xg = _sc_gather(xf, src.reshape(NWORK, NWCH, WCH))
    y = _grouped(xg, ew1, ew2, be)
    zw = _outproj(y, eow, eob, wg, be)
    routed2 = _sc_scatter(zw, dest.reshape(NWORK, NWCH, WCH))
    shared = _shared(xf, sw1, sw2)
    out = _combine(shared, routed2).reshape(B, S, DIM)
    return (out, lb[0, 0])
